# Initial kernel scaffold; baseline (speedup 1.0000x reference)
import jax
import jax.numpy as jnp
from jax.experimental import pallas as pl


def kernel(node_emb, rel_mat, W1, b1, W2, b2, generate_neighbor_emb, pos_edges, neg1_edges, neg2_edges):
    raise NotImplementedError("write your pallas kernel here")


# speedup vs baseline: 1.2780x; 1.2780x over previous
import functools

import jax
import jax.numpy as jnp
from jax import lax
from jax.experimental import pallas as pl
from jax.experimental.pallas import tpu as pltpu
from jax.experimental.pallas import tpu_sc as plsc

N_NODES = 100000
D = 572
N_REL = 6
E = 12000
E3 = 3 * E
E2 = 2 * E
N_SRC = N_REL * E3
N_DST = N_REL * E2
N_GATHER = N_SRC + N_DST
DP = 640

NW = 32
PER_W = 11264
N_PAD = NW * PER_W
CHUNK = 128
ITERS = PER_W // CHUNK


def _sc_gather(table, idx):
  mesh = plsc.VectorSubcoreMesh(core_axis_name="c", subcore_axis_name="s")

  @functools.partial(
      pl.kernel,
      out_type=jax.ShapeDtypeStruct((N_PAD, DP), jnp.float32),
      mesh=mesh,
      scratch_types=[
          pltpu.VMEM((ITERS, CHUNK), jnp.int32),
          pltpu.VMEM((CHUNK, DP), jnp.float32),
          pltpu.SemaphoreType.DMA,
      ],
  )
  def k(table_hbm, idx_hbm, out_hbm, idx_v, rows_v, sem):
    cid = lax.axis_index("c")
    sid = lax.axis_index("s")
    wid = sid * 2 + cid
    base = wid * PER_W
    pltpu.sync_copy(idx_hbm.at[wid], idx_v)

    def body(it, carry):
      pltpu.async_copy(table_hbm.at[idx_v.at[it]], rows_v, sem).wait()
      pltpu.sync_copy(rows_v, out_hbm.at[pl.ds(base + it * CHUNK, CHUNK)])
      return carry

    lax.fori_loop(0, ITERS, body, 0, unroll=False)

  return k(table, idx)


BE = 2000
NB = E3 // BE
NB_DST = E2 // BE


def _score_body(src_ref, dst_ref, fake_ref, rel_ref, out_ref):
  j = pl.program_id(1)
  s = jnp.dot(src_ref[0], rel_ref[0], preferred_element_type=jnp.float32)

  @pl.when(j < NB_DST)
  def _():
    out_ref[...] = jnp.sum(s * dst_ref[0], axis=1).reshape(1, 1, BE)

  @pl.when(j >= NB_DST)
  def _():
    out_ref[...] = jnp.sum(s[:, :D] * fake_ref[0], axis=1).reshape(1, 1, BE)


def _score_tc(src_rows, dst_rows, fake, rel_mat_p):
  out = pl.pallas_call(
      _score_body,
      grid=(N_REL, NB),
      in_specs=[
          pl.BlockSpec((1, BE, DP), lambda i, j: (i, j, 0)),
          pl.BlockSpec((1, BE, DP), lambda i, j: (i, jnp.minimum(j, NB_DST - 1), 0)),
          pl.BlockSpec((1, BE, D), lambda i, j: (i, jnp.maximum(j - NB_DST, 0), 0)),
          pl.BlockSpec((1, DP, DP), lambda i, j: (i, 0, 0)),
      ],
      out_specs=pl.BlockSpec((1, 1, BE), lambda i, j: (i * NB + j, 0, 0)),
      out_shape=jax.ShapeDtypeStruct((N_REL * NB, 1, BE), jnp.float32),
  )(src_rows, dst_rows, fake, rel_mat_p)
  return out.reshape(N_REL, E3)


BN = 2000
NBN = N_NODES // BN


def _graph_body(ne_ref, w1_ref, b1_ref, w2_ref, b2_ref, out_ref, acc_ref):
  k = pl.program_id(0)

  @pl.when(k == 0)
  def _():
    acc_ref[...] = jnp.zeros_like(acc_ref)

  acc_ref[...] += jnp.sum(ne_ref[...], axis=0, keepdims=True)

  @pl.when(k == NBN - 1)
  def _():
    hg = acc_ref[...] * jnp.float32(1.0 / N_NODES)
    h1 = jnp.maximum(
        jnp.dot(hg, w1_ref[...], preferred_element_type=jnp.float32)
        + b1_ref[...], 0.0)
    out_ref[...] = (
        jnp.dot(h1, w2_ref[...], preferred_element_type=jnp.float32)
        + b2_ref[...])


def _graph_tc(node_emb, w1, b1, w2, b2):
  out = pl.pallas_call(
      _graph_body,
      grid=(NBN,),
      in_specs=[
          pl.BlockSpec((BN, D), lambda k: (k, 0)),
          pl.BlockSpec((D, D // 2), lambda k: (0, 0)),
          pl.BlockSpec((1, D // 2), lambda k: (0, 0)),
          pl.BlockSpec((D // 2, 1), lambda k: (0, 0)),
          pl.BlockSpec((1, 1), lambda k: (0, 0)),
      ],
      out_specs=pl.BlockSpec((1, 1), lambda k: (0, 0)),
      out_shape=jax.ShapeDtypeStruct((1, 1), jnp.float32),
      scratch_shapes=[pltpu.VMEM((1, D), jnp.float32)],
  )(node_emb, w1, b1.reshape(1, -1), w2, b2.reshape(1, 1))
  return out.reshape(1)


def kernel(node_emb, rel_mat, W1, b1, W2, b2, generate_neighbor_emb,
           pos_edges, neg1_edges, neg2_edges):
  src_idx = jnp.concatenate(
      [pos_edges[:, 0], neg1_edges[:, 0], neg2_edges[:, 0]], axis=1)
  dst_idx = jnp.concatenate([pos_edges[:, 1], neg1_edges[:, 1]], axis=1)
  idx_all = jnp.concatenate([src_idx.reshape(-1), dst_idx.reshape(-1)])
  idx_pad = jnp.pad(idx_all, (0, N_PAD - N_GATHER)).reshape(NW, ITERS, CHUNK)

  node_emb_p = jnp.pad(node_emb, ((0, 0), (0, DP - D)))
  rel_mat_p = jnp.pad(rel_mat, ((0, 0), (0, DP - D), (0, DP - D)))

  gathered = _sc_gather(node_emb_p, idx_pad)
  src_rows = gathered[:N_SRC].reshape(N_REL, E3, DP)
  dst_rows = gathered[N_SRC:N_GATHER].reshape(N_REL, E2, DP)

  scores = _score_tc(src_rows, dst_rows, generate_neighbor_emb, rel_mat_p)
  graph_embd = _graph_tc(node_emb, W1, b1, W2, b2)

  pos_score = scores[:, :E].reshape(-1)
  neg_score1 = scores[:, E:2 * E].reshape(-1)
  neg_score2 = scores[:, 2 * E:].reshape(-1)
  return (pos_score, neg_score1, neg_score2, graph_embd)


# speedup vs baseline: 1.2900x; 1.0094x over previous
import functools

import jax
import jax.numpy as jnp
from jax import lax
from jax.experimental import pallas as pl
from jax.experimental.pallas import tpu as pltpu
from jax.experimental.pallas import tpu_sc as plsc

N_NODES = 100000
D = 572
N_REL = 6
E = 12000
E3 = 3 * E
E2 = 2 * E
N_SRC = N_REL * E3
N_DST = N_REL * E2
N_GATHER = N_SRC + N_DST
DP = 640

NW = 32
PER_W = 11264
N_PAD = NW * PER_W
CHUNK = 88
ITERS = PER_W // CHUNK


def _sc_gather(table, idx):
  mesh = plsc.VectorSubcoreMesh(core_axis_name="c", subcore_axis_name="s")

  @functools.partial(
      pl.kernel,
      out_type=jax.ShapeDtypeStruct((N_PAD, DP), jnp.float32),
      mesh=mesh,
      scratch_types=[
          pltpu.VMEM((ITERS, CHUNK), jnp.int32),
          pltpu.VMEM((CHUNK, DP), jnp.float32),
          pltpu.VMEM((CHUNK, DP), jnp.float32),
          pltpu.SemaphoreType.DMA,
          pltpu.SemaphoreType.DMA,
      ],
  )
  def k(table_hbm, idx_hbm, out_hbm, idx_v, rows0, rows1, sem0, sem1):
    cid = lax.axis_index("c")
    sid = lax.axis_index("s")
    wid = sid * 2 + cid
    base = wid * PER_W
    pltpu.sync_copy(idx_hbm.at[wid], idx_v)

    def gather(it, buf, sem):
      return pltpu.async_copy(table_hbm.at[idx_v.at[it]], buf, sem)

    def put(it, buf):
      pltpu.sync_copy(buf, out_hbm.at[pl.ds(base + it * CHUNK, CHUNK)])

    gather(0, rows0, sem0)

    def body(hk, carry):
      it0 = 2 * hk
      pltpu.make_async_copy(table_hbm.at[idx_v.at[it0]], rows0, sem0).wait()
      gather(it0 + 1, rows1, sem1)
      put(it0, rows0)
      pltpu.make_async_copy(table_hbm.at[idx_v.at[it0 + 1]], rows1, sem1).wait()

      @pl.when(it0 + 2 < ITERS)
      def _():
        gather(it0 + 2, rows0, sem0)

      put(it0 + 1, rows1)
      return carry

    lax.fori_loop(0, ITERS // 2, body, 0, unroll=False)

  return k(table, idx)


BE = 2000
NB = E3 // BE
NB_DST = E2 // BE


def _score_body(src_ref, dst_ref, fake_ref, rel_ref, out_ref):
  j = pl.program_id(1)
  s = jnp.dot(src_ref[0], rel_ref[0], preferred_element_type=jnp.float32)

  @pl.when(j < NB_DST)
  def _():
    out_ref[...] = jnp.sum(s * dst_ref[0], axis=1).reshape(1, 1, BE)

  @pl.when(j >= NB_DST)
  def _():
    out_ref[...] = jnp.sum(s[:, :D] * fake_ref[0], axis=1).reshape(1, 1, BE)


def _score_tc(src_rows, dst_rows, fake, rel_mat_p):
  out = pl.pallas_call(
      _score_body,
      grid=(N_REL, NB),
      in_specs=[
          pl.BlockSpec((1, BE, DP), lambda i, j: (i, j, 0)),
          pl.BlockSpec((1, BE, DP), lambda i, j: (i, jnp.minimum(j, NB_DST - 1), 0)),
          pl.BlockSpec((1, BE, D), lambda i, j: (i, jnp.maximum(j - NB_DST, 0), 0)),
          pl.BlockSpec((1, DP, DP), lambda i, j: (i, 0, 0)),
      ],
      out_specs=pl.BlockSpec((1, 1, BE), lambda i, j: (i * NB + j, 0, 0)),
      out_shape=jax.ShapeDtypeStruct((N_REL * NB, 1, BE), jnp.float32),
  )(src_rows, dst_rows, fake, rel_mat_p)
  return out.reshape(N_REL, E3)


BN = 2000
NBN = N_NODES // BN


def _graph_body(ne_ref, w1_ref, b1_ref, w2_ref, b2_ref, out_ref, acc_ref):
  k = pl.program_id(0)

  @pl.when(k == 0)
  def _():
    acc_ref[...] = jnp.zeros_like(acc_ref)

  acc_ref[...] += jnp.sum(ne_ref[...], axis=0, keepdims=True)

  @pl.when(k == NBN - 1)
  def _():
    hg = acc_ref[...] * jnp.float32(1.0 / N_NODES)
    hgb = hg.astype(jnp.bfloat16).astype(jnp.float32)
    w1b = w1_ref[...].astype(jnp.bfloat16).astype(jnp.float32)
    h1 = jnp.maximum(
        jnp.dot(hgb, w1b, preferred_element_type=jnp.float32,
                precision=lax.Precision.HIGHEST)
        + b1_ref[...], 0.0)
    out_ref[...] = (
        jnp.dot(h1, w2_ref[...], preferred_element_type=jnp.float32,
                precision=lax.Precision.HIGHEST)
        + b2_ref[...])


def _graph_tc(node_emb, w1, b1, w2, b2):
  out = pl.pallas_call(
      _graph_body,
      grid=(NBN,),
      in_specs=[
          pl.BlockSpec((BN, D), lambda k: (k, 0)),
          pl.BlockSpec((D, D // 2), lambda k: (0, 0)),
          pl.BlockSpec((1, D // 2), lambda k: (0, 0)),
          pl.BlockSpec((D // 2, 1), lambda k: (0, 0)),
          pl.BlockSpec((1, 1), lambda k: (0, 0)),
      ],
      out_specs=pl.BlockSpec((1, 1), lambda k: (0, 0)),
      out_shape=jax.ShapeDtypeStruct((1, 1), jnp.float32),
      scratch_shapes=[pltpu.VMEM((1, D), jnp.float32)],
  )(node_emb, w1, b1.reshape(1, -1), w2, b2.reshape(1, 1))
  return out.reshape(1)


def kernel(node_emb, rel_mat, W1, b1, W2, b2, generate_neighbor_emb,
           pos_edges, neg1_edges, neg2_edges):
  src_idx = jnp.concatenate(
      [pos_edges[:, 0], neg1_edges[:, 0], neg2_edges[:, 0]], axis=1)
  dst_idx = jnp.concatenate([pos_edges[:, 1], neg1_edges[:, 1]], axis=1)
  idx_all = jnp.concatenate([src_idx.reshape(-1), dst_idx.reshape(-1)])
  idx_pad = jnp.pad(idx_all, (0, N_PAD - N_GATHER)).reshape(NW, ITERS, CHUNK)

  node_emb_p = jnp.pad(node_emb, ((0, 0), (0, DP - D)))
  rel_mat_p = jnp.pad(rel_mat, ((0, 0), (0, DP - D), (0, DP - D)))

  gathered = _sc_gather(node_emb_p, idx_pad)
  src_rows = gathered[:N_SRC].reshape(N_REL, E3, DP)
  dst_rows = gathered[N_SRC:N_GATHER].reshape(N_REL, E2, DP)

  scores = _score_tc(src_rows, dst_rows, generate_neighbor_emb, rel_mat_p)
  graph_embd = _graph_tc(node_emb, W1, b1, W2, b2)

  pos_score = scores[:, :E].reshape(-1)
  neg_score1 = scores[:, E:2 * E].reshape(-1)
  neg_score2 = scores[:, 2 * E:].reshape(-1)
  return (pos_score, neg_score1, neg_score2, graph_embd)


# speedup vs baseline: 1.7963x; 1.3925x over previous
import functools

import jax
import jax.numpy as jnp
from jax import lax
from jax.experimental import pallas as pl
from jax.experimental.pallas import tpu as pltpu
from jax.experimental.pallas import tpu_sc as plsc

N_NODES = 100000
D = 572
N_REL = 6
E = 12000
E3 = 3 * E
E2 = 2 * E
N_SRC = N_REL * E3
N_DST = N_REL * E2
N_GATHER = N_SRC + N_DST
DP = 640

NW = 32
PER_W = 11264
N_PAD = NW * PER_W
CHUNK = 88
ITERS = PER_W // CHUNK


def _sc_gather(table, idx):
  mesh = plsc.VectorSubcoreMesh(core_axis_name="c", subcore_axis_name="s")

  @functools.partial(
      pl.kernel,
      out_type=jax.ShapeDtypeStruct((N_PAD, DP), jnp.float32),
      mesh=mesh,
      scratch_types=[
          pltpu.VMEM((ITERS, CHUNK), jnp.int32),
          pltpu.VMEM((CHUNK, DP), jnp.float32),
          pltpu.VMEM((CHUNK, DP), jnp.float32),
          pltpu.SemaphoreType.DMA,
          pltpu.SemaphoreType.DMA,
      ],
  )
  def k(table_hbm, idx_hbm, out_hbm, idx_v, rows0, rows1, sem0, sem1):
    cid = lax.axis_index("c")
    sid = lax.axis_index("s")
    wid = sid * 2 + cid
    base = wid * PER_W
    pltpu.sync_copy(idx_hbm.at[wid], idx_v)

    def gather(it, buf, sem):
      return pltpu.async_copy(table_hbm.at[idx_v.at[it]], buf, sem)

    def put(it, buf):
      pltpu.sync_copy(buf, out_hbm.at[pl.ds(base + it * CHUNK, CHUNK)])

    gather(0, rows0, sem0)

    def body(hk, carry):
      it0 = 2 * hk
      pltpu.make_async_copy(table_hbm.at[idx_v.at[it0]], rows0, sem0).wait()
      gather(it0 + 1, rows1, sem1)
      put(it0, rows0)
      pltpu.make_async_copy(table_hbm.at[idx_v.at[it0 + 1]], rows1, sem1).wait()

      @pl.when(it0 + 2 < ITERS)
      def _():
        gather(it0 + 2, rows0, sem0)

      put(it0 + 1, rows1)
      return carry

    lax.fori_loop(0, ITERS // 2, body, 0, unroll=False)

  return k(table, idx)


BE = 2000
NB = E3 // BE
NB_DST = E2 // BE


def _score_body(src_ref, dst_ref, fake_ref, rel_ref, out_ref):
  j = pl.program_id(1)
  s = jnp.dot(src_ref[...], rel_ref[0], preferred_element_type=jnp.float32)

  @pl.when(j < NB_DST)
  def _():
    out_ref[...] = jnp.sum(s * dst_ref[...], axis=1).reshape(1, 1, BE)

  @pl.when(j >= NB_DST)
  def _():
    out_ref[...] = jnp.sum(s[:, :D] * fake_ref[0], axis=1).reshape(1, 1, BE)


def _score_tc(gathered, fake, rel_mat_p):
  out = pl.pallas_call(
      _score_body,
      grid=(N_REL, NB),
      in_specs=[
          pl.BlockSpec((BE, DP), lambda i, j: (i * NB + j, 0)),
          pl.BlockSpec(
              (BE, DP),
              lambda i, j: (N_SRC // BE + i * NB_DST + jnp.minimum(j, NB_DST - 1), 0)),
          pl.BlockSpec((1, BE, D), lambda i, j: (i, jnp.maximum(j - NB_DST, 0), 0)),
          pl.BlockSpec((1, DP, DP), lambda i, j: (i, 0, 0)),
      ],
      out_specs=pl.BlockSpec((1, 1, BE), lambda i, j: (i * NB + j, 0, 0)),
      out_shape=jax.ShapeDtypeStruct((N_REL * NB, 1, BE), jnp.float32),
  )(gathered, gathered, fake, rel_mat_p)
  return out.reshape(N_REL, E3)


BN = 2000
NBN = N_NODES // BN


def _graph_body(ne_ref, w1_ref, b1_ref, w2_ref, b2_ref, out_ref, acc_ref):
  k = pl.program_id(0)

  @pl.when(k == 0)
  def _():
    acc_ref[...] = jnp.zeros_like(acc_ref)

  acc_ref[...] += jnp.sum(ne_ref[...], axis=0, keepdims=True)

  @pl.when(k == NBN - 1)
  def _():
    hg = acc_ref[...] * jnp.float32(1.0 / N_NODES)
    hgb = hg.astype(jnp.bfloat16).astype(jnp.float32)
    w1b = w1_ref[...].astype(jnp.bfloat16).astype(jnp.float32)
    h1 = jnp.maximum(
        jnp.dot(hgb, w1b, preferred_element_type=jnp.float32,
                precision=lax.Precision.HIGHEST)
        + b1_ref[...], 0.0)
    out_ref[...] = (
        jnp.dot(h1, w2_ref[...], preferred_element_type=jnp.float32,
                precision=lax.Precision.HIGHEST)
        + b2_ref[...])


def _graph_tc(node_emb, w1, b1, w2, b2):
  out = pl.pallas_call(
      _graph_body,
      grid=(NBN,),
      in_specs=[
          pl.BlockSpec((BN, D), lambda k: (k, 0)),
          pl.BlockSpec((D, D // 2), lambda k: (0, 0)),
          pl.BlockSpec((1, D // 2), lambda k: (0, 0)),
          pl.BlockSpec((D // 2, 1), lambda k: (0, 0)),
          pl.BlockSpec((1, 1), lambda k: (0, 0)),
      ],
      out_specs=pl.BlockSpec((1, 1), lambda k: (0, 0)),
      out_shape=jax.ShapeDtypeStruct((1, 1), jnp.float32),
      scratch_shapes=[pltpu.VMEM((1, D), jnp.float32)],
  )(node_emb, w1, b1.reshape(1, -1), w2, b2.reshape(1, 1))
  return out.reshape(1)


def kernel(node_emb, rel_mat, W1, b1, W2, b2, generate_neighbor_emb,
           pos_edges, neg1_edges, neg2_edges):
  src_idx = jnp.concatenate(
      [pos_edges[:, 0], neg1_edges[:, 0], neg2_edges[:, 0]], axis=1)
  dst_idx = jnp.concatenate([pos_edges[:, 1], neg1_edges[:, 1]], axis=1)
  idx_all = jnp.concatenate([src_idx.reshape(-1), dst_idx.reshape(-1)])
  idx_pad = jnp.pad(idx_all, (0, N_PAD - N_GATHER)).reshape(NW, ITERS, CHUNK)

  node_emb_p = jnp.pad(node_emb, ((0, 0), (0, DP - D)))
  rel_mat_p = jnp.pad(rel_mat, ((0, 0), (0, DP - D), (0, DP - D)))

  gathered = _sc_gather(node_emb_p, idx_pad)
  scores = _score_tc(gathered, generate_neighbor_emb, rel_mat_p)
  graph_embd = _graph_tc(node_emb, W1, b1, W2, b2)

  pos_score = scores[:, :E].reshape(-1)
  neg_score1 = scores[:, E:2 * E].reshape(-1)
  neg_score2 = scores[:, 2 * E:].reshape(-1)
  return (pos_score, neg_score1, neg_score2, graph_embd)
